# Initial kernel scaffold; baseline (speedup 1.0000x reference)
import jax
import jax.numpy as jnp
from jax.experimental import pallas as pl


def kernel(x, edge_index, edge_attr, u, batch, params):
    raise NotImplementedError("write your pallas kernel here")


# speedup vs baseline: 4.8380x; 4.8380x over previous
import functools

import jax
import jax.numpy as jnp
from jax import lax
from jax.experimental import pallas as pl
from jax.experimental.pallas import tpu as pltpu
from jax.experimental.pallas import tpu_sc as plsc

F32 = jnp.float32

_N = 10000
_E = 320000
_NC = 2
_NS = 16
_NW = _NC * _NS
_CH = 80
_EPW = _E // _NW
_NCH = _EPW // _CH
_RPS = _N // _NS
_RE = 2000
_RN = 1000
_PREC = lax.Precision.DEFAULT


def _mesh():
    return plsc.VectorSubcoreMesh(
        core_axis_name="c", subcore_axis_name="s",
        num_cores=_NC, num_subcores=_NS)


def _sc_gather(xs, xd, src3, dst3):
    @functools.partial(
        pl.kernel,
        out_type=jax.ShapeDtypeStruct((_E, 128), F32),
        mesh=_mesh(),
        compiler_params=pltpu.CompilerParams(use_tc_tiling_on_sc=False),
        scratch_types=[
            pltpu.VMEM((_NCH, _CH), jnp.int32),
            pltpu.VMEM((_NCH, _CH), jnp.int32),
            pltpu.VMEM((_CH, 128), F32),
            pltpu.VMEM((_CH, 128), F32),
            pltpu.SemaphoreType.DMA,
            pltpu.SemaphoreType.DMA,
        ],
    )
    def k(xs_hbm, xd_hbm, src_hbm, dst_hbm, out_hbm, isv, idv, bs, bd, s1, s2):
        c = lax.axis_index("c")
        s = lax.axis_index("s")
        w = s * _NC + c
        pltpu.sync_copy(src_hbm.at[w], isv)
        pltpu.sync_copy(dst_hbm.at[w], idv)
        base = w * _EPW

        def chunk(j, carry):
            cps = pltpu.async_copy(xs_hbm.at[isv.at[j]], bs, s1)
            cpd = pltpu.async_copy(xd_hbm.at[idv.at[j]], bd, s2)
            cps.wait()
            cpd.wait()

            def row(r, carry2):
                for cc in range(8):
                    sl = pl.ds(cc * 16, 16)
                    bs[r, sl] = bs[r, sl] + bd[r, sl]
                return carry2

            lax.fori_loop(0, _CH, row, 0)
            pltpu.sync_copy(bs, out_hbm.at[pl.ds(base + j * _CH, _CH)])
            return carry

        lax.fori_loop(0, _NCH, chunk, 0)

    return k(xs, xd, src3, dst3)


def _sc_scatter(e, dst3):
    @functools.partial(
        pl.kernel,
        out_type=jax.ShapeDtypeStruct((_NC, _N, 128), F32),
        mesh=_mesh(),
        compiler_params=pltpu.CompilerParams(use_tc_tiling_on_sc=False),
        scratch_types=[
            pltpu.VMEM((_NCH, _CH), jnp.int32),
            pltpu.VMEM((_CH, 128), F32),
            pltpu.VMEM((25, 128), F32),
            pltpu.VMEM_SHARED((_N, 128), F32),
            pltpu.SemaphoreType.DMA,
        ],
    )
    def k(e_hbm, dst_hbm, out_hbm, idv, ebuf, zbuf, acc, sem):
        c = lax.axis_index("c")
        s = lax.axis_index("s")
        w = s * _NC + c
        pltpu.sync_copy(dst_hbm.at[w], idv)
        z16 = jnp.zeros((16,), F32)
        for r in range(25):
            for cc in range(8):
                zbuf[r, pl.ds(cc * 16, 16)] = z16

        def zrow(j, carry):
            pltpu.sync_copy(zbuf, acc.at[pl.ds(s * _RPS + j * 25, 25)])
            return carry

        lax.fori_loop(0, _RPS // 25, zrow, 0)
        plsc.subcore_barrier()
        base = w * _EPW

        def chunk(j, carry):
            pltpu.sync_copy(e_hbm.at[pl.ds(base + j * _CH, _CH)], ebuf)
            pltpu.sync_copy(ebuf, acc.at[idv.at[j]], add=True)
            return carry

        lax.fori_loop(0, _NCH, chunk, 0)
        plsc.subcore_barrier()
        pltpu.sync_copy(acc.at[pl.ds(s * _RPS, _RPS)],
                        out_hbm.at[c, pl.ds(s * _RPS, _RPS)])

    return k(e, dst3)


def _sc_deg(dst3):
    @functools.partial(
        pl.kernel,
        out_type=jax.ShapeDtypeStruct((_NC, _N, 16), F32),
        mesh=_mesh(),
        compiler_params=pltpu.CompilerParams(use_tc_tiling_on_sc=False),
        scratch_types=[
            pltpu.VMEM((_NCH, _CH), jnp.int32),
            pltpu.VMEM((_CH, 16), F32),
            pltpu.VMEM((25, 16), F32),
            pltpu.VMEM_SHARED((_N, 16), F32),
        ],
    )
    def k(dst_hbm, out_hbm, idv, obuf, zbuf, acc):
        c = lax.axis_index("c")
        s = lax.axis_index("s")
        w = s * _NC + c
        pltpu.sync_copy(dst_hbm.at[w], idv)
        o16 = jnp.full((16,), 1.0, F32)
        z16 = jnp.zeros((16,), F32)
        for r in range(_CH):
            obuf[r, pl.ds(0, 16)] = o16
        for r in range(25):
            zbuf[r, pl.ds(0, 16)] = z16

        def zrow(j, carry):
            pltpu.sync_copy(zbuf, acc.at[pl.ds(s * _RPS + j * 25, 25)])
            return carry

        lax.fori_loop(0, _RPS // 25, zrow, 0)
        plsc.subcore_barrier()

        def chunk(j, carry):
            pltpu.sync_copy(obuf, acc.at[idv.at[j]], add=True)
            return carry

        lax.fori_loop(0, _NCH, chunk, 0)
        plsc.subcore_barrier()
        pltpu.sync_copy(acc.at[pl.ds(s * _RPS, _RPS)],
                        out_hbm.at[c, pl.ds(s * _RPS, _RPS)])

    return k(dst3)


def _tc_proj(Xc, A, B, Uc, D, b1, Dn, b1n):
    dn = Xc.shape[1]
    dg = Uc.shape[1]
    grid = _N // _RN

    def body(xc_ref, a_ref, b_ref, uc_ref, d_ref, b1_ref, dn_ref, b1n_ref,
             xs_ref, xd_ref, ue_ref, un_ref):
        xc = xc_ref[...]
        xs_ref[...] = jnp.dot(xc, a_ref[...], precision=_PREC,
                              preferred_element_type=F32)
        xd_ref[...] = jnp.dot(xc, b_ref[...], precision=_PREC,
                              preferred_element_type=F32)

        @pl.when(pl.program_id(0) == 0)
        def _():
            uc = uc_ref[...]
            ue_ref[...] = jnp.dot(uc, d_ref[...], precision=_PREC,
                                  preferred_element_type=F32) + b1_ref[...]
            un_ref[...] = jnp.dot(uc, dn_ref[...], precision=_PREC,
                                  preferred_element_type=F32) + b1n_ref[...]

    full = lambda shp: pl.BlockSpec(shp, lambda i: (0, 0))
    return pl.pallas_call(
        body,
        grid=(grid,),
        in_specs=[
            pl.BlockSpec((_RN, dn), lambda i: (i, 0)),
            full((dn, 128)), full((dn, 128)),
            full((1, dg)), full((dg, 128)), full((1, 128)),
            full((dg, 128)), full((1, 128)),
        ],
        out_specs=[
            pl.BlockSpec((_RN, 128), lambda i: (i, 0)),
            pl.BlockSpec((_RN, 128), lambda i: (i, 0)),
            full((1, 128)), full((1, 128)),
        ],
        out_shape=[
            jax.ShapeDtypeStruct((_N, 128), F32),
            jax.ShapeDtypeStruct((_N, 128), F32),
            jax.ShapeDtypeStruct((1, 128), F32),
            jax.ShapeDtypeStruct((1, 128), F32),
        ],
    )(Xc, A, B, Uc, D, b1, Dn, b1n)


def _tc_edge(g, Elist, Cs, w2, urow, b2, gam, bet, act, norm):
    ne = len(Elist)
    grid = _E // _RE

    def body(*refs):
        g_ref = refs[0]
        e_refs = refs[1:1 + ne]
        c_refs = refs[1 + ne:1 + 2 * ne]
        w2_ref, ue_ref, b2_ref, gam_ref, bet_ref = refs[1 + 2 * ne:6 + 2 * ne]
        out_ref, esum_ref = refs[6 + 2 * ne:]
        h = g_ref[...] + ue_ref[...]
        for er, cr in zip(e_refs, c_refs):
            h = h + jnp.dot(er[...], cr[...], precision=_PREC,
                            preferred_element_type=F32)
        h = jnp.maximum(h, 0.0)
        e = jnp.dot(h, w2_ref[...], precision=_PREC,
                    preferred_element_type=F32) + b2_ref[...]
        if act:
            e = jnp.maximum(e, 0.0)
        if norm:
            m = jnp.mean(e, axis=1, keepdims=True)
            v = jnp.mean((e - m) ** 2, axis=1, keepdims=True)
            e = (e - m) / jnp.sqrt(v + 1e-5) * gam_ref[...] + bet_ref[...]
        out_ref[...] = e

        @pl.when(pl.program_id(0) == 0)
        def _():
            esum_ref[...] = jnp.zeros_like(esum_ref)

        esum_ref[...] += jnp.sum(e, axis=0, keepdims=True)

    full = lambda shp: pl.BlockSpec(shp, lambda i: (0, 0))
    in_specs = ([pl.BlockSpec((_RE, 128), lambda i: (i, 0))]
                + [pl.BlockSpec((_RE, ej.shape[1]), lambda i: (i, 0))
                   for ej in Elist]
                + [full(c.shape) for c in Cs]
                + [full((128, 128)), full((1, 128)), full((1, 128)),
                   full((1, 128)), full((1, 128))])
    return pl.pallas_call(
        body,
        grid=(grid,),
        in_specs=in_specs,
        out_specs=[
            pl.BlockSpec((_RE, 128), lambda i: (i, 0)),
            full((1, 128)),
        ],
        out_shape=[
            jax.ShapeDtypeStruct((_E, 128), F32),
            jax.ShapeDtypeStruct((1, 128), F32),
        ],
    )(g, *Elist, *Cs, w2, urow, b2, gam, bet)


def _tc_node(Xc, aggs, degs, P, Q, urow, w2, b2, gam, bet, act, norm):
    dn = Xc.shape[1]
    grid = _N // _RN

    def body(xc_ref, ag_ref, dg_ref, p_ref, q_ref, ur_ref, w2_ref, b2_ref,
             gam_ref, bet_ref, out_ref, xsum_ref):
        agg = ag_ref[0] + ag_ref[1]
        deg = jnp.sum(dg_ref[0] + dg_ref[1], axis=1, keepdims=True) * (1.0 / 16.0)
        scale = 1.0 / jnp.maximum(deg, 1.0)
        h = jnp.dot(xc_ref[...], p_ref[...], precision=_PREC,
                    preferred_element_type=F32)
        h = h + jnp.dot(agg * scale, q_ref[...], precision=_PREC,
                        preferred_element_type=F32)
        h = jnp.maximum(h + ur_ref[...], 0.0)
        xn = jnp.dot(h, w2_ref[...], precision=_PREC,
                     preferred_element_type=F32) + b2_ref[...]
        if act:
            xn = jnp.maximum(xn, 0.0)
        if norm:
            m = jnp.mean(xn, axis=1, keepdims=True)
            v = jnp.mean((xn - m) ** 2, axis=1, keepdims=True)
            xn = (xn - m) / jnp.sqrt(v + 1e-5) * gam_ref[...] + bet_ref[...]
        out_ref[...] = xn

        @pl.when(pl.program_id(0) == 0)
        def _():
            xsum_ref[...] = jnp.zeros_like(xsum_ref)

        xsum_ref[...] += jnp.sum(xn, axis=0, keepdims=True)

    full = lambda shp: pl.BlockSpec(shp, lambda i: (0, 0))
    return pl.pallas_call(
        body,
        grid=(grid,),
        in_specs=[
            pl.BlockSpec((_RN, dn), lambda i: (i, 0)),
            pl.BlockSpec((_NC, _RN, 128), lambda i: (0, i, 0)),
            pl.BlockSpec((_NC, _RN, 16), lambda i: (0, i, 0)),
            full((dn, 128)), full((128, 128)), full((1, 128)),
            full((128, 128)), full((1, 128)), full((1, 128)), full((1, 128)),
        ],
        out_specs=[
            pl.BlockSpec((_RN, 128), lambda i: (i, 0)),
            full((1, 128)),
        ],
        out_shape=[
            jax.ShapeDtypeStruct((_N, 128), F32),
            jax.ShapeDtypeStruct((1, 128), F32),
        ],
    )(Xc, aggs, degs, P, Q, urow, w2, b2, gam, bet)


def _tc_glob(Uc, xnsum, esum, Gu, Gn, Ge, b1, w2, b2, act):
    dg = Uc.shape[1]

    def body(uc_ref, xs_ref, es_ref, gu_ref, gn_ref, ge_ref, b1_ref,
             w2_ref, b2_ref, out_ref):
        h = jnp.dot(uc_ref[...], gu_ref[...], precision=_PREC,
                    preferred_element_type=F32)
        h = h + jnp.dot(xs_ref[...] * (1.0 / _N), gn_ref[...], precision=_PREC,
                        preferred_element_type=F32)
        h = h + jnp.dot(es_ref[...] * (1.0 / _E), ge_ref[...], precision=_PREC,
                        preferred_element_type=F32)
        h = jnp.maximum(h + b1_ref[...], 0.0)
        un = jnp.dot(h, w2_ref[...], precision=_PREC,
                     preferred_element_type=F32) + b2_ref[...]
        if act:
            un = jnp.maximum(un, 0.0)
        out_ref[...] = un

    full = lambda shp: pl.BlockSpec(shp, lambda i: (0, 0))
    return pl.pallas_call(
        body,
        grid=(1,),
        in_specs=[
            full((1, dg)), full((1, 128)), full((1, 128)),
            full((dg, 128)), full((128, 128)), full((128, 128)),
            full((1, 128)), full((128, 128)), full((1, 128)),
        ],
        out_specs=full((1, 128)),
        out_shape=jax.ShapeDtypeStruct((1, 128), F32),
    )(Uc, xnsum, esum, Gu, Gn, Ge, b1, w2, b2)


def _pad_lat(w, lat):
    return jnp.pad(w, ((0, 0), (0, 128 - lat))) if lat < 128 else w


def _prep(p, dn, des, dg, lat):
    pe, pn, pg = p["edge"], p["node"], p["global"]
    w1 = pe["w1"]
    A = _pad_lat(w1[:dn], lat)
    B = _pad_lat(w1[dn:2 * dn], lat)
    Cs, off = [], 2 * dn
    for de in des:
        Cs.append(_pad_lat(w1[off:off + de], lat))
        off += de
    D = _pad_lat(w1[off:off + dg], lat)
    b1 = _pad_lat(pe["b1"][None, :], lat)
    w2 = jnp.pad(pe["w2"], ((0, 128 - lat), (0, 0))) if lat < 128 else pe["w2"]
    n1 = pn["w1"]
    P = _pad_lat(n1[:dn], lat)
    Q = _pad_lat(n1[dn:dn + 128], lat)
    Dn = _pad_lat(n1[dn + 128:], lat)
    b1n = _pad_lat(pn["b1"][None, :], lat)
    w2n = jnp.pad(pn["w2"], ((0, 128 - lat), (0, 0))) if lat < 128 else pn["w2"]
    g1 = pg["w1"]
    Gu = _pad_lat(g1[:dg], lat)
    Gn = _pad_lat(g1[dg:dg + 128], lat)
    Ge = _pad_lat(g1[dg + 128:], lat)
    b1g = _pad_lat(pg["b1"][None, :], lat)
    w2g = jnp.pad(pg["w2"], ((0, 128 - lat), (0, 0))) if lat < 128 else pg["w2"]

    def row(name, d):
        return d[name][None, :] if name in d else jnp.zeros((1, 128), F32)

    return dict(
        A=A, B=B, Cs=Cs, D=D, b1=b1, w2=w2, b2=pe["b2"][None, :],
        ge=row("g", pe), be=row("bn", pe),
        P=P, Q=Q, Dn=Dn, b1n=b1n, w2n=w2n, b2n=pn["b2"][None, :],
        gn=row("g", pn), bn=row("bn", pn),
        Gu=Gu, Gn=Gn, Ge=Ge, b1g=b1g, w2g=w2g, b2g=pg["b2"][None, :],
    )


def _gn_block(Xc, Elist, Uc, src3, dst3, degs, w, act, norm):
    xs, xd, urow_e, urow_n = _tc_proj(
        Xc, w["A"], w["B"], Uc, w["D"], w["b1"], w["Dn"], w["b1n"])
    g = _sc_gather(xs, xd, src3, dst3)
    e, esum = _tc_edge(g, Elist, w["Cs"], w["w2"], urow_e, w["b2"],
                       w["ge"], w["be"], act, norm)
    aggs = _sc_scatter(e, dst3)
    xn, xnsum = _tc_node(Xc, aggs, degs, w["P"], w["Q"], urow_n,
                         w["w2n"], w["b2n"], w["gn"], w["bn"], act, norm)
    un = _tc_glob(Uc, xnsum, esum, w["Gu"], w["Gn"], w["Ge"],
                  w["b1g"], w["w2g"], w["b2g"], act)
    return xn, e, un


def kernel(x, edge_index, edge_attr, u, batch, params):
    del batch
    src3 = edge_index[0].reshape(_NW, _NCH, _CH)
    dst3 = edge_index[1].reshape(_NW, _NCH, _CH)
    degs = _sc_deg(dst3)

    we = _prep(params["encoder"], 128, [16], 16, 128)
    wp = _prep(params["processor"], 256, [128, 128], 256, 128)
    wd = _prep(params["decoder"], 128, [128], 128, 128)
    wo = _prep(params["output"], 128, [128], 128, 65)

    x0, e0, u0 = _gn_block(x, [edge_attr], u, src3, dst3, degs, we,
                           True, True)
    xc, ec, uc = x0, e0, u0
    for _ in range(5):
        Xc = jnp.concatenate([x0, xc], axis=1)
        Uc = jnp.concatenate([u0, uc], axis=1)
        xc, ec, uc = _gn_block(Xc, [e0, ec], Uc, src3, dst3, degs, wp,
                               True, True)
    dx, de, du = _gn_block(xc, [ec], uc, src3, dst3, degs, wd, True, True)
    out = _gn_block(dx, [de], du, src3, dst3, degs, wo, False, False)
    return out


# speedup vs baseline: 5.6215x; 1.1619x over previous
import functools

import jax
import jax.numpy as jnp
from jax import lax
from jax.experimental import pallas as pl
from jax.experimental.pallas import tpu as pltpu
from jax.experimental.pallas import tpu_sc as plsc

F32 = jnp.float32

_N = 10000
_E = 320000
_NC = 2
_NS = 16
_NW = _NC * _NS
_CH = 80
_EPW = _E // _NW
_NCH = _EPW // _CH
_RPS = _N // _NS
_RE = 2000
_RN = 1000
_PREC = lax.Precision.DEFAULT


def _mesh():
    return plsc.VectorSubcoreMesh(
        core_axis_name="c", subcore_axis_name="s",
        num_cores=_NC, num_subcores=_NS)


def _sc_gather(xs, xd, src3, dst3):
    @functools.partial(
        pl.kernel,
        out_type=jax.ShapeDtypeStruct((_E, 128), F32),
        mesh=_mesh(),
        compiler_params=pltpu.CompilerParams(use_tc_tiling_on_sc=False),
        scratch_types=[
            pltpu.VMEM((_NCH, _CH), jnp.int32),
            pltpu.VMEM((_NCH, _CH), jnp.int32),
            pltpu.VMEM((_CH, 128), F32),
            pltpu.VMEM((_CH, 128), F32),
            pltpu.VMEM((_CH, 128), F32),
            pltpu.VMEM((_CH, 128), F32),
            pltpu.SemaphoreType.DMA,
            pltpu.SemaphoreType.DMA,
            pltpu.SemaphoreType.DMA,
            pltpu.SemaphoreType.DMA,
        ],
    )
    def k(xs_hbm, xd_hbm, src_hbm, dst_hbm, out_hbm, isv, idv,
          bsA, bdA, bsB, bdB, gsA, gsB, osA, osB):
        c = lax.axis_index("c")
        s = lax.axis_index("s")
        w = s * _NC + c
        pltpu.sync_copy(src_hbm.at[w], isv)
        pltpu.sync_copy(dst_hbm.at[w], idv)
        base = w * _EPW

        def issue(j, bs, bd, sem):
            pltpu.async_copy(xs_hbm.at[isv.at[j]], bs, sem)
            pltpu.async_copy(xd_hbm.at[idv.at[j]], bd, sem)

        def gwait(bs, bd, sem):
            pltpu.make_async_copy(xs_hbm.at[isv.at[0]], bs, sem).wait()
            pltpu.make_async_copy(xd_hbm.at[idv.at[0]], bd, sem).wait()

        def owait(bs, sem):
            pltpu.make_async_copy(bs, out_hbm.at[pl.ds(0, _CH)], sem).wait()

        def add(bs, bd):
            def row(r, carry2):
                for cc in range(8):
                    sl = pl.ds(cc * 16, 16)
                    bs[r, sl] = bs[r, sl] + bd[r, sl]
                return carry2

            lax.fori_loop(0, _CH, row, 0)

        issue(0, bsA, bdA, gsA)

        def body(i, carry):
            j0 = 2 * i

            @pl.when(i > 0)
            def _():
                owait(bsB, osB)

            gwait(bsA, bdA, gsA)

            @pl.when(j0 + 1 < _NCH)
            def _():
                issue(j0 + 1, bsB, bdB, gsB)

            add(bsA, bdA)
            pltpu.async_copy(bsA, out_hbm.at[pl.ds(base + j0 * _CH, _CH)], osA)

            @pl.when(j0 + 1 < _NCH)
            def _():
                gwait(bsB, bdB, gsB)
                add(bsB, bdB)
                pltpu.async_copy(
                    bsB, out_hbm.at[pl.ds(base + (j0 + 1) * _CH, _CH)], osB)

                @pl.when(j0 + 2 < _NCH)
                def _():
                    owait(bsA, osA)
                    issue(j0 + 2, bsA, bdA, gsA)

            return carry

        lax.fori_loop(0, (_NCH + 1) // 2, body, 0)
        owait(bsA, osA)

    return k(xs, xd, src3, dst3)


def _sc_scatter(e, dst3):
    @functools.partial(
        pl.kernel,
        out_type=jax.ShapeDtypeStruct((_NC, _N, 128), F32),
        mesh=_mesh(),
        compiler_params=pltpu.CompilerParams(use_tc_tiling_on_sc=False),
        scratch_types=[
            pltpu.VMEM((_NCH, _CH), jnp.int32),
            pltpu.VMEM((_CH, 128), F32),
            pltpu.VMEM((_CH, 128), F32),
            pltpu.VMEM((25, 128), F32),
            pltpu.VMEM_SHARED((_N, 128), F32),
            pltpu.SemaphoreType.DMA,
            pltpu.SemaphoreType.DMA,
        ],
    )
    def k(e_hbm, dst_hbm, out_hbm, idv, ebA, ebB, zbuf, acc, sA, sB):
        c = lax.axis_index("c")
        s = lax.axis_index("s")
        w = s * _NC + c
        pltpu.sync_copy(dst_hbm.at[w], idv)
        z16 = jnp.zeros((16,), F32)
        for r in range(25):
            for cc in range(8):
                zbuf[r, pl.ds(cc * 16, 16)] = z16

        def zrow(j, carry):
            pltpu.sync_copy(zbuf, acc.at[pl.ds(s * _RPS + j * 25, 25)])
            return carry

        lax.fori_loop(0, _RPS // 25, zrow, 0)
        plsc.subcore_barrier()
        base = w * _EPW

        def issue(j, eb, sem):
            pltpu.async_copy(e_hbm.at[pl.ds(base + j * _CH, _CH)], eb, sem)

        def rwait(eb, sem):
            pltpu.make_async_copy(e_hbm.at[pl.ds(0, _CH)], eb, sem).wait()

        issue(0, ebA, sA)

        def body(i, carry):
            j0 = 2 * i
            rwait(ebA, sA)

            @pl.when(j0 + 1 < _NCH)
            def _():
                issue(j0 + 1, ebB, sB)

            pltpu.sync_copy(ebA, acc.at[idv.at[j0]], add=True)

            @pl.when(j0 + 1 < _NCH)
            def _():
                rwait(ebB, sB)

                @pl.when(j0 + 2 < _NCH)
                def _():
                    issue(j0 + 2, ebA, sA)

                pltpu.sync_copy(ebB, acc.at[idv.at[j0 + 1]], add=True)

            return carry

        lax.fori_loop(0, (_NCH + 1) // 2, body, 0)
        plsc.subcore_barrier()
        pltpu.sync_copy(acc.at[pl.ds(s * _RPS, _RPS)],
                        out_hbm.at[c, pl.ds(s * _RPS, _RPS)])

    return k(e, dst3)


def _sc_deg(dst3):
    @functools.partial(
        pl.kernel,
        out_type=jax.ShapeDtypeStruct((_NC, _N, 16), F32),
        mesh=_mesh(),
        compiler_params=pltpu.CompilerParams(use_tc_tiling_on_sc=False),
        scratch_types=[
            pltpu.VMEM((_NCH, _CH), jnp.int32),
            pltpu.VMEM((_CH, 16), F32),
            pltpu.VMEM((25, 16), F32),
            pltpu.VMEM_SHARED((_N, 16), F32),
        ],
    )
    def k(dst_hbm, out_hbm, idv, obuf, zbuf, acc):
        c = lax.axis_index("c")
        s = lax.axis_index("s")
        w = s * _NC + c
        pltpu.sync_copy(dst_hbm.at[w], idv)
        o16 = jnp.full((16,), 1.0, F32)
        z16 = jnp.zeros((16,), F32)
        for r in range(_CH):
            obuf[r, pl.ds(0, 16)] = o16
        for r in range(25):
            zbuf[r, pl.ds(0, 16)] = z16

        def zrow(j, carry):
            pltpu.sync_copy(zbuf, acc.at[pl.ds(s * _RPS + j * 25, 25)])
            return carry

        lax.fori_loop(0, _RPS // 25, zrow, 0)
        plsc.subcore_barrier()

        def chunk(j, carry):
            pltpu.sync_copy(obuf, acc.at[idv.at[j]], add=True)
            return carry

        lax.fori_loop(0, _NCH, chunk, 0)
        plsc.subcore_barrier()
        pltpu.sync_copy(acc.at[pl.ds(s * _RPS, _RPS)],
                        out_hbm.at[c, pl.ds(s * _RPS, _RPS)])

    return k(dst3)


def _tc_proj(Xc, A, B, Uc, D, b1, Dn, b1n):
    dn = Xc.shape[1]
    dg = Uc.shape[1]
    grid = _N // _RN

    def body(xc_ref, a_ref, b_ref, uc_ref, d_ref, b1_ref, dn_ref, b1n_ref,
             xs_ref, xd_ref, ue_ref, un_ref):
        xc = xc_ref[...]
        xs_ref[...] = jnp.dot(xc, a_ref[...], precision=_PREC,
                              preferred_element_type=F32)
        xd_ref[...] = jnp.dot(xc, b_ref[...], precision=_PREC,
                              preferred_element_type=F32)

        @pl.when(pl.program_id(0) == 0)
        def _():
            uc = uc_ref[...]
            ue_ref[...] = jnp.dot(uc, d_ref[...], precision=_PREC,
                                  preferred_element_type=F32) + b1_ref[...]
            un_ref[...] = jnp.dot(uc, dn_ref[...], precision=_PREC,
                                  preferred_element_type=F32) + b1n_ref[...]

    full = lambda shp: pl.BlockSpec(shp, lambda i: (0, 0))
    return pl.pallas_call(
        body,
        grid=(grid,),
        in_specs=[
            pl.BlockSpec((_RN, dn), lambda i: (i, 0)),
            full((dn, 128)), full((dn, 128)),
            full((1, dg)), full((dg, 128)), full((1, 128)),
            full((dg, 128)), full((1, 128)),
        ],
        out_specs=[
            pl.BlockSpec((_RN, 128), lambda i: (i, 0)),
            pl.BlockSpec((_RN, 128), lambda i: (i, 0)),
            full((1, 128)), full((1, 128)),
        ],
        out_shape=[
            jax.ShapeDtypeStruct((_N, 128), F32),
            jax.ShapeDtypeStruct((_N, 128), F32),
            jax.ShapeDtypeStruct((1, 128), F32),
            jax.ShapeDtypeStruct((1, 128), F32),
        ],
    )(Xc, A, B, Uc, D, b1, Dn, b1n)


def _tc_edge(g, Elist, Cs, w2, urow, b2, gam, bet, act, norm):
    ne = len(Elist)
    grid = _E // _RE

    def body(*refs):
        g_ref = refs[0]
        e_refs = refs[1:1 + ne]
        c_refs = refs[1 + ne:1 + 2 * ne]
        w2_ref, ue_ref, b2_ref, gam_ref, bet_ref = refs[1 + 2 * ne:6 + 2 * ne]
        out_ref, esum_ref = refs[6 + 2 * ne:]
        h = g_ref[...] + ue_ref[...]
        for er, cr in zip(e_refs, c_refs):
            h = h + jnp.dot(er[...], cr[...], precision=_PREC,
                            preferred_element_type=F32)
        h = jnp.maximum(h, 0.0)
        e = jnp.dot(h, w2_ref[...], precision=_PREC,
                    preferred_element_type=F32) + b2_ref[...]
        if act:
            e = jnp.maximum(e, 0.0)
        if norm:
            m = jnp.mean(e, axis=1, keepdims=True)
            v = jnp.mean((e - m) ** 2, axis=1, keepdims=True)
            e = (e - m) / jnp.sqrt(v + 1e-5) * gam_ref[...] + bet_ref[...]
        out_ref[...] = e

        @pl.when(pl.program_id(0) == 0)
        def _():
            esum_ref[...] = jnp.zeros_like(esum_ref)

        esum_ref[...] += jnp.sum(e, axis=0, keepdims=True)

    full = lambda shp: pl.BlockSpec(shp, lambda i: (0, 0))
    in_specs = ([pl.BlockSpec((_RE, 128), lambda i: (i, 0))]
                + [pl.BlockSpec((_RE, ej.shape[1]), lambda i: (i, 0))
                   for ej in Elist]
                + [full(c.shape) for c in Cs]
                + [full((128, 128)), full((1, 128)), full((1, 128)),
                   full((1, 128)), full((1, 128))])
    return pl.pallas_call(
        body,
        grid=(grid,),
        in_specs=in_specs,
        out_specs=[
            pl.BlockSpec((_RE, 128), lambda i: (i, 0)),
            full((1, 128)),
        ],
        out_shape=[
            jax.ShapeDtypeStruct((_E, 128), F32),
            jax.ShapeDtypeStruct((1, 128), F32),
        ],
    )(g, *Elist, *Cs, w2, urow, b2, gam, bet)


def _tc_node(Xc, aggs, degs, P, Q, urow, w2, b2, gam, bet, act, norm):
    dn = Xc.shape[1]
    grid = _N // _RN

    def body(xc_ref, ag_ref, dg_ref, p_ref, q_ref, ur_ref, w2_ref, b2_ref,
             gam_ref, bet_ref, out_ref, xsum_ref):
        agg = ag_ref[0] + ag_ref[1]
        deg = jnp.sum(dg_ref[0] + dg_ref[1], axis=1, keepdims=True) * (1.0 / 16.0)
        scale = 1.0 / jnp.maximum(deg, 1.0)
        h = jnp.dot(xc_ref[...], p_ref[...], precision=_PREC,
                    preferred_element_type=F32)
        h = h + jnp.dot(agg * scale, q_ref[...], precision=_PREC,
                        preferred_element_type=F32)
        h = jnp.maximum(h + ur_ref[...], 0.0)
        xn = jnp.dot(h, w2_ref[...], precision=_PREC,
                     preferred_element_type=F32) + b2_ref[...]
        if act:
            xn = jnp.maximum(xn, 0.0)
        if norm:
            m = jnp.mean(xn, axis=1, keepdims=True)
            v = jnp.mean((xn - m) ** 2, axis=1, keepdims=True)
            xn = (xn - m) / jnp.sqrt(v + 1e-5) * gam_ref[...] + bet_ref[...]
        out_ref[...] = xn

        @pl.when(pl.program_id(0) == 0)
        def _():
            xsum_ref[...] = jnp.zeros_like(xsum_ref)

        xsum_ref[...] += jnp.sum(xn, axis=0, keepdims=True)

    full = lambda shp: pl.BlockSpec(shp, lambda i: (0, 0))
    return pl.pallas_call(
        body,
        grid=(grid,),
        in_specs=[
            pl.BlockSpec((_RN, dn), lambda i: (i, 0)),
            pl.BlockSpec((_NC, _RN, 128), lambda i: (0, i, 0)),
            pl.BlockSpec((_NC, _RN, 16), lambda i: (0, i, 0)),
            full((dn, 128)), full((128, 128)), full((1, 128)),
            full((128, 128)), full((1, 128)), full((1, 128)), full((1, 128)),
        ],
        out_specs=[
            pl.BlockSpec((_RN, 128), lambda i: (i, 0)),
            full((1, 128)),
        ],
        out_shape=[
            jax.ShapeDtypeStruct((_N, 128), F32),
            jax.ShapeDtypeStruct((1, 128), F32),
        ],
    )(Xc, aggs, degs, P, Q, urow, w2, b2, gam, bet)


def _tc_glob(Uc, xnsum, esum, Gu, Gn, Ge, b1, w2, b2, act):
    dg = Uc.shape[1]

    def body(uc_ref, xs_ref, es_ref, gu_ref, gn_ref, ge_ref, b1_ref,
             w2_ref, b2_ref, out_ref):
        h = jnp.dot(uc_ref[...], gu_ref[...], precision=_PREC,
                    preferred_element_type=F32)
        h = h + jnp.dot(xs_ref[...] * (1.0 / _N), gn_ref[...], precision=_PREC,
                        preferred_element_type=F32)
        h = h + jnp.dot(es_ref[...] * (1.0 / _E), ge_ref[...], precision=_PREC,
                        preferred_element_type=F32)
        h = jnp.maximum(h + b1_ref[...], 0.0)
        un = jnp.dot(h, w2_ref[...], precision=_PREC,
                     preferred_element_type=F32) + b2_ref[...]
        if act:
            un = jnp.maximum(un, 0.0)
        out_ref[...] = un

    full = lambda shp: pl.BlockSpec(shp, lambda i: (0, 0))
    return pl.pallas_call(
        body,
        grid=(1,),
        in_specs=[
            full((1, dg)), full((1, 128)), full((1, 128)),
            full((dg, 128)), full((128, 128)), full((128, 128)),
            full((1, 128)), full((128, 128)), full((1, 128)),
        ],
        out_specs=full((1, 128)),
        out_shape=jax.ShapeDtypeStruct((1, 128), F32),
    )(Uc, xnsum, esum, Gu, Gn, Ge, b1, w2, b2)


def _pad_lat(w, lat):
    return jnp.pad(w, ((0, 0), (0, 128 - lat))) if lat < 128 else w


def _prep(p, dn, des, dg, lat):
    pe, pn, pg = p["edge"], p["node"], p["global"]
    w1 = pe["w1"]
    A = _pad_lat(w1[:dn], lat)
    B = _pad_lat(w1[dn:2 * dn], lat)
    Cs, off = [], 2 * dn
    for de in des:
        Cs.append(_pad_lat(w1[off:off + de], lat))
        off += de
    D = _pad_lat(w1[off:off + dg], lat)
    b1 = _pad_lat(pe["b1"][None, :], lat)
    w2 = jnp.pad(pe["w2"], ((0, 128 - lat), (0, 0))) if lat < 128 else pe["w2"]
    n1 = pn["w1"]
    P = _pad_lat(n1[:dn], lat)
    Q = _pad_lat(n1[dn:dn + 128], lat)
    Dn = _pad_lat(n1[dn + 128:], lat)
    b1n = _pad_lat(pn["b1"][None, :], lat)
    w2n = jnp.pad(pn["w2"], ((0, 128 - lat), (0, 0))) if lat < 128 else pn["w2"]
    g1 = pg["w1"]
    Gu = _pad_lat(g1[:dg], lat)
    Gn = _pad_lat(g1[dg:dg + 128], lat)
    Ge = _pad_lat(g1[dg + 128:], lat)
    b1g = _pad_lat(pg["b1"][None, :], lat)
    w2g = jnp.pad(pg["w2"], ((0, 128 - lat), (0, 0))) if lat < 128 else pg["w2"]

    def row(name, d):
        return d[name][None, :] if name in d else jnp.zeros((1, 128), F32)

    return dict(
        A=A, B=B, Cs=Cs, D=D, b1=b1, w2=w2, b2=pe["b2"][None, :],
        ge=row("g", pe), be=row("bn", pe),
        P=P, Q=Q, Dn=Dn, b1n=b1n, w2n=w2n, b2n=pn["b2"][None, :],
        gn=row("g", pn), bn=row("bn", pn),
        Gu=Gu, Gn=Gn, Ge=Ge, b1g=b1g, w2g=w2g, b2g=pg["b2"][None, :],
    )


def _gn_block(Xc, Elist, Uc, src3, dst3, degs, w, act, norm):
    xs, xd, urow_e, urow_n = _tc_proj(
        Xc, w["A"], w["B"], Uc, w["D"], w["b1"], w["Dn"], w["b1n"])
    g = _sc_gather(xs, xd, src3, dst3)
    e, esum = _tc_edge(g, Elist, w["Cs"], w["w2"], urow_e, w["b2"],
                       w["ge"], w["be"], act, norm)
    aggs = _sc_scatter(e, dst3)
    xn, xnsum = _tc_node(Xc, aggs, degs, w["P"], w["Q"], urow_n,
                         w["w2n"], w["b2n"], w["gn"], w["bn"], act, norm)
    un = _tc_glob(Uc, xnsum, esum, w["Gu"], w["Gn"], w["Ge"],
                  w["b1g"], w["w2g"], w["b2g"], act)
    return xn, e, un


def kernel(x, edge_index, edge_attr, u, batch, params):
    del batch
    src3 = edge_index[0].reshape(_NW, _NCH, _CH)
    dst3 = edge_index[1].reshape(_NW, _NCH, _CH)
    degs = _sc_deg(dst3)

    we = _prep(params["encoder"], 128, [16], 16, 128)
    wp = _prep(params["processor"], 256, [128, 128], 256, 128)
    wd = _prep(params["decoder"], 128, [128], 128, 128)
    wo = _prep(params["output"], 128, [128], 128, 65)

    x0, e0, u0 = _gn_block(x, [edge_attr], u, src3, dst3, degs, we,
                           True, True)
    xc, ec, uc = x0, e0, u0
    for _ in range(5):
        Xc = jnp.concatenate([x0, xc], axis=1)
        Uc = jnp.concatenate([u0, uc], axis=1)
        xc, ec, uc = _gn_block(Xc, [e0, ec], Uc, src3, dst3, degs, wp,
                               True, True)
    dx, de, du = _gn_block(xc, [ec], uc, src3, dst3, degs, wd, True, True)
    out = _gn_block(dx, [de], du, src3, dst3, degs, wo, False, False)
    return out
